# baseline (speedup 1.0000x reference)
import functools
import jax
import jax.numpy as jnp
from jax import lax
from jax.experimental import pallas as pl
from jax.experimental.pallas import tpu as pltpu
from jax.experimental.pallas import tpu_sc as plsc

_D = 64
_SCALE = 8.0
_NC = 2
_NS = 16
_NW = _NC * _NS
_CHUNK = 128
_LANES = 16


def _make_emb_kernel(n_chunks: int):
  b_per_w = n_chunks * _CHUNK
  total_b = b_per_w * _NW
  mesh = plsc.VectorSubcoreMesh(core_axis_name="c", subcore_axis_name="s",
                                num_cores=_NC, num_subcores=_NS)

  @functools.partial(
      pl.kernel,
      mesh=mesh,
      compiler_params=pltpu.CompilerParams(use_tc_tiling_on_sc=False),
      out_type=jax.ShapeDtypeStruct((total_b, _D), jnp.float32),
      scratch_types=[
          pltpu.VMEM((n_chunks, _CHUNK), jnp.int32),
          pltpu.VMEM((_CHUNK, _D), jnp.float32),
          pltpu.SemaphoreType.DMA,
      ],
  )
  def emb(tokens_hbm, table_hbm, out_hbm, idx_v, rows_v, sem):
    wid = lax.axis_index("s") * _NC + lax.axis_index("c")
    base = wid * b_per_w
    pltpu.sync_copy(tokens_hbm.at[wid], idx_v)

    def chunk_body(j, carry):
      pltpu.async_copy(table_hbm.at[idx_v.at[j]], rows_v, sem).wait()

      def row_body(r, c2):
        for c in range(_D // _LANES):
          sl = pl.ds(c * _LANES, _LANES)
          rows_v[r, sl] = rows_v[r, sl] * _SCALE
        return c2

      lax.fori_loop(0, _CHUNK, row_body, 0, unroll=2)
      pltpu.sync_copy(rows_v, out_hbm.at[pl.ds(base + j * _CHUNK, _CHUNK)])
      return carry

    lax.fori_loop(0, n_chunks, chunk_body, 0)

  return emb


def kernel(tokens, table):
  bt, seq = tokens.shape
  total = bt * seq
  n_chunks = total // (_NW * _CHUNK)
  tokens_flat = tokens.reshape(_NW, n_chunks, _CHUNK).astype(jnp.int32)
  out = _make_emb_kernel(n_chunks)(tokens_flat, table)
  return out.reshape(bt, seq, _D)
